# baseline (speedup 1.0000x reference)
import jax
import jax.numpy as jnp
from jax.experimental import pallas as pl
from jax.experimental.pallas import tpu as pltpu


def _moe_mm_kernel(bias_ref, x_ref, w_ref, o_ref):
    for i in range(x_ref.shape[0]):
        acc = jax.lax.dot_general(
            x_ref[i], w_ref[i], (((1,), (1,)), ((), ())),
            preferred_element_type=jnp.float32,
        )
        o_ref[i] = acc + bias_ref[0]


def kernel(inputs, moe_weight, expert_size):
    num_experts, output_size, input_size = moe_weight.shape
    total_tokens = inputs.shape[0]
    tokens_per_expert = total_tokens // num_experts

    x = inputs.reshape(num_experts, tokens_per_expert, input_size)
    bias = (
        jnp.asarray(expert_size, jnp.float32) - jnp.float32(tokens_per_expert)
    ).reshape(1)

    experts_per_step = 4
    num_steps = num_experts // experts_per_step
    out = pl.pallas_call(
        _moe_mm_kernel,
        grid=(num_steps,),
        in_specs=[
            pl.BlockSpec(memory_space=pltpu.SMEM),
            pl.BlockSpec(
                (experts_per_step, tokens_per_expert, input_size),
                lambda e: (e, 0, 0),
            ),
            pl.BlockSpec(
                (experts_per_step, output_size, input_size), lambda e: (e, 0, 0)
            ),
        ],
        out_specs=pl.BlockSpec(
            (experts_per_step, tokens_per_expert, output_size), lambda e: (e, 0, 0)
        ),
        out_shape=jax.ShapeDtypeStruct(
            (num_experts, tokens_per_expert, output_size), jnp.float32
        ),
        compiler_params=pltpu.CompilerParams(dimension_semantics=("parallel",), vmem_limit_bytes=100 * 1024 * 1024),
    )(bias, x, moe_weight)
    return out.reshape(total_tokens, output_size)
